# Initial kernel scaffold; baseline (speedup 1.0000x reference)
import jax
import jax.numpy as jnp
from jax.experimental import pallas as pl


def kernel(input_batch, emb_table, W1, b1, W2, b2):
    raise NotImplementedError("write your pallas kernel here")


# speedup vs baseline: 2.2779x; 2.2779x over previous
import functools

import jax
import jax.numpy as jnp
from jax import lax
from jax.experimental import pallas as pl
from jax.experimental.pallas import tpu as pltpu
from jax.experimental.pallas import tpu_sc as plsc

_VOCAB = 1000000
_EMB = 64
_HID = 128
_B = 16384
_L = 50


_ROWS_BLK = 8000
_NBLK = _VOCAB // _ROWS_BLK


def _score_body(w1_ref, b1_ref, w2_ref, b2_ref, table_ref, s_ref):
    w2 = w2_ref[...]
    w = jnp.dot(w2, w1_ref[...], preferred_element_type=jnp.float32,
                precision=lax.Precision.HIGHEST)
    c = jnp.sum(w2 * b1_ref[...]) + b2_ref[0, 0]
    t = table_ref[...]
    s = lax.dot_general(w, t, (((1,), (1,)), ((), ())),
                        preferred_element_type=jnp.float32,
                        precision=lax.Precision.HIGHEST)
    s_ref[...] = ((s + c) * (1.0 / _L)).reshape(1, 1, _ROWS_BLK)


_score = pl.pallas_call(
    _score_body,
    grid=(_NBLK,),
    in_specs=[
        pl.BlockSpec((_HID, _EMB), lambda i: (0, 0)),
        pl.BlockSpec((1, _HID), lambda i: (0, 0)),
        pl.BlockSpec((1, _HID), lambda i: (0, 0)),
        pl.BlockSpec((1, 1), lambda i: (0, 0)),
        pl.BlockSpec((_ROWS_BLK, _EMB), lambda i: (i, 0)),
    ],
    out_specs=pl.BlockSpec((1, 1, _ROWS_BLK), lambda i: (i, 0, 0)),
    out_shape=jax.ShapeDtypeStruct((_NBLK, 1, _ROWS_BLK), jnp.float32),
)


_NC, _NS, _LANES = 2, 16, 16
_NW = _NC * _NS
_TOK_W = (_B * _L) // _NW
_BAG_W = _B // _NW
_GRP = _BAG_W // _LANES

_sc_mesh = plsc.VectorSubcoreMesh(core_axis_name="c", subcore_axis_name="s")


@functools.partial(
    pl.kernel,
    mesh=_sc_mesh,
    out_type=jax.ShapeDtypeStruct((_B,), jnp.float32),
    compiler_params=pltpu.CompilerParams(needs_layout_passes=False),
    scratch_types=[
        pltpu.VMEM((_TOK_W,), jnp.int32),
        pltpu.VMEM((_TOK_W,), jnp.float32),
        pltpu.VMEM((_BAG_W,), jnp.float32),
        pltpu.SemaphoreType.DMA,
    ],
)
def _pool(s_hbm, idx_hbm, out_hbm, idx_v, val_v, out_v, sem):
    wid = lax.axis_index("s") * _NC + lax.axis_index("c")
    tbase = wid * _TOK_W
    pltpu.sync_copy(idx_hbm.at[pl.ds(tbase, _TOK_W)], idx_v)
    pltpu.async_copy(s_hbm.at[idx_v], val_v, sem).wait()
    lane = jnp.arange(_LANES, dtype=jnp.int32) * _L

    def body(g, carry):
        base = lane + g * (_LANES * _L)
        acc = plsc.load_gather(val_v, [base])
        for l in range(1, _L):
            acc = acc + plsc.load_gather(val_v, [base + l])
        out_v[pl.ds(g * _LANES, _LANES)] = acc
        return carry

    lax.fori_loop(0, _GRP, body, 0)
    pltpu.sync_copy(out_v, out_hbm.at[pl.ds(wid * _BAG_W, _BAG_W)])


def kernel(input_batch, emb_table, W1, b1, W2, b2):
    s = _score(W1, b1.reshape(1, _HID), W2, b2.reshape(1, 1), emb_table)
    out = _pool(s.reshape(_VOCAB), input_batch.reshape(_B * _L))
    return out.reshape(_B, 1)


# speedup vs baseline: 3.5377x; 1.5530x over previous
import functools

import jax
import jax.numpy as jnp
from jax import lax
from jax.experimental import pallas as pl
from jax.experimental.pallas import tpu as pltpu
from jax.experimental.pallas import tpu_sc as plsc

_VOCAB = 1000000
_EMB = 64
_HID = 128
_B = 16384
_L = 50


_ROWS_BLK = 25000
_NBLK = _VOCAB // _ROWS_BLK


def _score_body(w1_ref, b1_ref, w2_ref, b2_ref, table_ref, s_ref):
    w2 = w2_ref[...]
    w = jnp.dot(w2, w1_ref[...], preferred_element_type=jnp.float32,
                precision=lax.Precision.HIGHEST)
    c = jnp.sum(w2 * b1_ref[...]) + b2_ref[0, 0]
    t = table_ref[...]
    s = lax.dot_general(w, t, (((1,), (1,)), ((), ())),
                        preferred_element_type=jnp.float32)
    s_ref[...] = ((s + c) * (1.0 / _L)).reshape(1, 1, _ROWS_BLK)


_score = pl.pallas_call(
    _score_body,
    grid=(_NBLK,),
    in_specs=[
        pl.BlockSpec((_HID, _EMB), lambda i: (0, 0)),
        pl.BlockSpec((1, _HID), lambda i: (0, 0)),
        pl.BlockSpec((1, _HID), lambda i: (0, 0)),
        pl.BlockSpec((1, 1), lambda i: (0, 0)),
        pl.BlockSpec((_ROWS_BLK, _EMB), lambda i: (i, 0)),
    ],
    out_specs=pl.BlockSpec((1, 1, _ROWS_BLK), lambda i: (i, 0, 0)),
    out_shape=jax.ShapeDtypeStruct((_NBLK, 1, _ROWS_BLK), jnp.float32),
)


_NC, _NS, _LANES = 2, 16, 16
_NW = _NC * _NS
_TOK_W = (_B * _L) // _NW
_BAG_W = _B // _NW
_GRP = _BAG_W // _LANES

_sc_mesh = plsc.VectorSubcoreMesh(core_axis_name="c", subcore_axis_name="s")


@functools.partial(
    pl.kernel,
    mesh=_sc_mesh,
    out_type=jax.ShapeDtypeStruct((_B,), jnp.float32),
    compiler_params=pltpu.CompilerParams(needs_layout_passes=False),
    scratch_types=[
        pltpu.VMEM((_TOK_W,), jnp.int32),
        pltpu.VMEM((_TOK_W,), jnp.float32),
        pltpu.VMEM((_BAG_W,), jnp.float32),
        pltpu.SemaphoreType.DMA,
    ],
)
def _pool(s_hbm, idx_hbm, out_hbm, idx_v, val_v, out_v, sem):
    wid = lax.axis_index("s") * _NC + lax.axis_index("c")
    tbase = wid * _TOK_W
    pltpu.sync_copy(idx_hbm.at[pl.ds(tbase, _TOK_W)], idx_v)
    pltpu.async_copy(s_hbm.at[idx_v], val_v, sem).wait()
    lane = jnp.arange(_LANES, dtype=jnp.int32) * _L

    def body(g, carry):
        base = lane + g * (_LANES * _L)
        acc = plsc.load_gather(val_v, [base])
        for l in range(1, _L):
            acc = acc + plsc.load_gather(val_v, [base + l])
        out_v[pl.ds(g * _LANES, _LANES)] = acc
        return carry

    lax.fori_loop(0, _GRP, body, 0)
    pltpu.sync_copy(out_v, out_hbm.at[pl.ds(wid * _BAG_W, _BAG_W)])


def kernel(input_batch, emb_table, W1, b1, W2, b2):
    s = _score(W1, b1.reshape(1, _HID), W2, b2.reshape(1, 1), emb_table)
    out = _pool(s.reshape(_VOCAB), input_batch.reshape(_B * _L))
    return out.reshape(_B, 1)


# speedup vs baseline: 3.5690x; 1.0088x over previous
import functools

import jax
import jax.numpy as jnp
from jax import lax
from jax.experimental import pallas as pl
from jax.experimental.pallas import tpu as pltpu
from jax.experimental.pallas import tpu_sc as plsc

_VOCAB = 1000000
_EMB = 64
_HID = 128
_B = 16384
_L = 50


_ROWS_BLK = 10000
_NSLICE = 4
_NBLK = _VOCAB // (_ROWS_BLK * _NSLICE)


def _score_body(w1_ref, b1_ref, w2_ref, b2_ref, *refs):
    table_refs, s_ref = refs[:_NSLICE], refs[_NSLICE]
    w2 = w2_ref[...]
    w = jnp.dot(w2, w1_ref[...], preferred_element_type=jnp.float32,
                precision=lax.Precision.HIGHEST)
    c = jnp.sum(w2 * b1_ref[...]) + b2_ref[0, 0]
    for k in range(_NSLICE):
        t = table_refs[k][...]
        s = lax.dot_general(w, t, (((1,), (1,)), ((), ())),
                            preferred_element_type=jnp.float32)
        s_ref[:, k:k + 1, :] = ((s + c) * (1.0 / _L)).reshape(1, 1, _ROWS_BLK)


def _table_spec(k):
    return pl.BlockSpec((_ROWS_BLK, _EMB), lambda i, k=k: (i * _NSLICE + k, 0))


_score = pl.pallas_call(
    _score_body,
    grid=(_NBLK,),
    in_specs=[
        pl.BlockSpec((_HID, _EMB), lambda i: (0, 0)),
        pl.BlockSpec((1, _HID), lambda i: (0, 0)),
        pl.BlockSpec((1, _HID), lambda i: (0, 0)),
        pl.BlockSpec((1, 1), lambda i: (0, 0)),
    ] + [_table_spec(k) for k in range(_NSLICE)],
    out_specs=pl.BlockSpec((1, _NSLICE, _ROWS_BLK), lambda i: (i, 0, 0)),
    out_shape=jax.ShapeDtypeStruct((_NBLK, _NSLICE, _ROWS_BLK), jnp.float32),
)


_NC, _NS, _LANES = 2, 16, 16
_NW = _NC * _NS
_TOK_W = (_B * _L) // _NW
_BAG_W = _B // _NW
_GRP = _BAG_W // _LANES

_sc_mesh = plsc.VectorSubcoreMesh(core_axis_name="c", subcore_axis_name="s")


@functools.partial(
    pl.kernel,
    mesh=_sc_mesh,
    out_type=jax.ShapeDtypeStruct((_B,), jnp.float32),
    compiler_params=pltpu.CompilerParams(needs_layout_passes=False),
    scratch_types=[
        pltpu.VMEM((_TOK_W,), jnp.int32),
        pltpu.VMEM((_TOK_W,), jnp.float32),
        pltpu.VMEM((_BAG_W,), jnp.float32),
        pltpu.SemaphoreType.DMA,
    ],
)
def _pool(s_hbm, idx_hbm, out_hbm, idx_v, val_v, out_v, sem):
    wid = lax.axis_index("s") * _NC + lax.axis_index("c")
    tbase = wid * _TOK_W
    pltpu.sync_copy(idx_hbm.at[pl.ds(tbase, _TOK_W)], idx_v)
    pltpu.async_copy(s_hbm.at[idx_v], val_v, sem).wait()
    lane = jnp.arange(_LANES, dtype=jnp.int32) * _L

    def body(g, carry):
        base = lane + g * (_LANES * _L)
        acc = plsc.load_gather(val_v, [base])
        for l in range(1, _L):
            acc = acc + plsc.load_gather(val_v, [base + l])
        out_v[pl.ds(g * _LANES, _LANES)] = acc
        return carry

    lax.fori_loop(0, _GRP, body, 0)
    pltpu.sync_copy(out_v, out_hbm.at[pl.ds(wid * _BAG_W, _BAG_W)])


def kernel(input_batch, emb_table, W1, b1, W2, b2):
    s = _score(W1, b1.reshape(1, _HID), W2, b2.reshape(1, 1),
               *([emb_table] * _NSLICE))
    out = _pool(s.reshape(_VOCAB), input_batch.reshape(_B * _L))
    return out.reshape(_B, 1)
